# baseline (speedup 1.0000x reference)
import jax
import jax.numpy as jnp
from jax.experimental import pallas as pl
from jax.experimental.pallas import tpu as pltpu

F32 = jnp.float32


def _first_argmax_mask(y):
    n = y.shape[0]
    it = jax.lax.broadcasted_iota(jnp.int32, y.shape, 0)
    m1 = jnp.max(y, axis=0, keepdims=True)
    a = jnp.min(jnp.where(y == m1, it, n), axis=0, keepdims=True)
    return (it == a).astype(F32)


def _top2_masks(y):
    n = y.shape[0]
    it = jax.lax.broadcasted_iota(jnp.int32, y.shape, 0)
    m1 = jnp.max(y, axis=0, keepdims=True)
    a = jnp.min(jnp.where(y == m1, it, n), axis=0, keepdims=True)
    h1 = (it == a).astype(F32)
    y2 = jnp.where(it == a, -jnp.inf, y)
    m2 = jnp.max(y2, axis=0, keepdims=True)
    b = jnp.min(jnp.where(y2 == m2, it, n), axis=0, keepdims=True)
    h2 = (it == b).astype(F32)
    return h1, h2


def _row_mask(m, n):
    it = jax.lax.broadcasted_iota(jnp.int32, (n, 1), 0)
    return (it == m).astype(F32)


def _routing_kernel(inp0_ref, loc0_ref, out_ref):
    e = inp0_ref[0]
    it = jax.lax.broadcasted_iota(jnp.int32, e.shape, 0)
    mx = jnp.max(e, axis=0, keepdims=True)
    c = jnp.min(jnp.where(e == mx, it, e.shape[0]), axis=0, keepdims=True)
    l = loc0_ref[0]
    it2 = jax.lax.broadcasted_iota(jnp.int32, l.shape, 0)
    mx2 = jnp.max(l, axis=0, keepdims=True)
    m = jnp.min(jnp.where(l == mx2, it2, l.shape[0]), axis=0, keepdims=True)
    out_ref[:] = jnp.concatenate([c, m], axis=0)


def _module_mlp(v1, v2, pm, mw1, mb1, mw2, mb2):
    pm3 = pm[:, :, None]
    w1s = jnp.sum(mw1 * pm3, axis=0)
    b1s = jnp.sum(mb1 * pm, axis=0, keepdims=True)
    w2s = jnp.sum(mw2 * pm3, axis=0)
    b2s = jnp.sum(mb2 * pm, axis=0, keepdims=True)
    h = jnp.maximum(v1 * w1s[0:1, :] + v2 * w1s[1:2, :] + b1s, 0.0)
    return jnp.dot(h, w2s, preferred_element_type=F32) + b2s


def _main_kernel(cm_ref, x_ref, w1_ref, b1_ref, w2_ref, b2_ref, w3_ref, b3_ref,
                 mw1_ref, mb1_ref, mw2_ref, mb2_ref,
                 ie1_ref, ie2_ref, ie3_ref, le1_ref, le2_ref,
                 out_ref, acc_ref):
    i = pl.program_id(0)
    bsz = x_ref.shape[0]

    @pl.when(i < 4)
    def _():
        m = cm_ref[4 + i]
        bm = _row_mask(m, 4)
        b1s = jnp.sum(b1_ref[:] * bm, axis=0, keepdims=True)
        b2s = jnp.sum(b2_ref[:] * bm, axis=0, keepdims=True)
        b3s = jnp.sum(b3_ref[:] * bm, axis=0, keepdims=True)
        flat = x_ref[:, 0].reshape(bsz, 784)
        h1 = jnp.maximum(
            jnp.dot(flat, w1_ref[0], preferred_element_type=F32) + b1s, 0.0)
        h2 = jnp.maximum(
            jnp.dot(h1, w2_ref[0], preferred_element_type=F32) + b2s, 0.0)
        y = jnp.dot(h2, w3_ref[0], preferred_element_type=F32) + b3s
        acc_ref[pl.ds(i, 1)] = y.reshape(1, bsz, 16)

    @pl.when(i == 4)
    def _():
        acc = acc_ref[:]
        ie1 = ie1_ref[0]
        le1 = le1_ref[0]
        mw1 = mw1_ref[:]
        mb1 = mb1_ref[:]
        mw2 = mw2_ref[:]
        mb2 = mb2_ref[:]
        cols2 = []
        for si in range(4):
            h1m, h2m = _top2_masks(jax.nn.sigmoid(ie1[:, si:si + 1]))
            h1r = h1m.reshape(4, 16)[:, None, :]
            h2r = h2m.reshape(4, 16)[:, None, :]
            v1 = jnp.sum(acc * h1r, axis=(0, 2))[:, None]
            v2 = jnp.sum(acc * h2r, axis=(0, 2))[:, None]
            pm = _first_argmax_mask(le1[:, si:si + 1])
            cols2.append(_module_mlp(v1, v2, pm, mw1, mb1, mw2, mb2))
        xc2 = jnp.concatenate(cols2, axis=1)
        ie2 = ie2_ref[0]
        le2 = le2_ref[0]
        cols3 = []
        for si in range(2):
            h1m, h2m = _top2_masks(jax.nn.sigmoid(ie2[:, si:si + 1]))
            v1 = jnp.sum(xc2 * h1m.reshape(1, 4), axis=1, keepdims=True)
            v2 = jnp.sum(xc2 * h2m.reshape(1, 4), axis=1, keepdims=True)
            pm = _first_argmax_mask(le2[:, si:si + 1])
            cols3.append(_module_mlp(v1, v2, pm, mw1, mb1, mw2, mb2))
        xc3 = jnp.concatenate(cols3, axis=1)
        h1m, h2m = _top2_masks(jax.nn.sigmoid(ie3_ref[0]))
        v1 = jnp.sum(xc3 * h1m.reshape(1, 2), axis=1, keepdims=True)
        v2 = jnp.sum(xc3 * h2m.reshape(1, 2), axis=1, keepdims=True)
        out_ref[:] = jax.nn.sigmoid(jnp.concatenate([v1, v2], axis=1))


def kernel(x, img_W1, img_b1, img_W2, img_b2, img_W3, img_b3,
           mod_W1, mod_b1, mod_W2, mod_b2,
           inp_emb0, inp_emb1, inp_emb2, inp_emb3,
           loc_emb0, loc_emb1, loc_emb2):
    bsz = x.shape[0]
    cm2 = pl.pallas_call(
        _routing_kernel,
        out_shape=jax.ShapeDtypeStruct((2, 4), jnp.int32),
    )(inp_emb0, loc_emb0)
    cm = cm2.reshape(8)

    def msel(i, cmr):
        return cmr[4 + jnp.minimum(i, 3)]

    def csel(i, cmr):
        return cmr[jnp.minimum(i, 3)]

    grid_spec = pltpu.PrefetchScalarGridSpec(
        num_scalar_prefetch=1,
        grid=(5,),
        in_specs=[
            pl.BlockSpec((bsz, 1, 28, 28), lambda i, cmr: (0, csel(i, cmr), 0, 0)),
            pl.BlockSpec((1, 784, 512), lambda i, cmr: (msel(i, cmr), 0, 0)),
            pl.BlockSpec((4, 512), lambda i, cmr: (0, 0)),
            pl.BlockSpec((1, 512, 512), lambda i, cmr: (msel(i, cmr), 0, 0)),
            pl.BlockSpec((4, 512), lambda i, cmr: (0, 0)),
            pl.BlockSpec((1, 512, 16), lambda i, cmr: (msel(i, cmr), 0, 0)),
            pl.BlockSpec((4, 16), lambda i, cmr: (0, 0)),
            pl.BlockSpec((8, 2, 128), lambda i, cmr: (0, 0, 0)),
            pl.BlockSpec((8, 128), lambda i, cmr: (0, 0)),
            pl.BlockSpec((8, 128, 1), lambda i, cmr: (0, 0, 0)),
            pl.BlockSpec((8, 1), lambda i, cmr: (0, 0)),
            pl.BlockSpec((1, 64, 4), lambda i, cmr: (0, 0, 0)),
            pl.BlockSpec((1, 4, 2), lambda i, cmr: (0, 0, 0)),
            pl.BlockSpec((1, 2, 1), lambda i, cmr: (0, 0, 0)),
            pl.BlockSpec((1, 8, 4), lambda i, cmr: (0, 0, 0)),
            pl.BlockSpec((1, 8, 2), lambda i, cmr: (0, 0, 0)),
        ],
        out_specs=pl.BlockSpec((bsz, 2), lambda i, cmr: (0, 0)),
        scratch_shapes=[
            pltpu.VMEM((4, bsz, 16), F32),
        ],
    )
    out = pl.pallas_call(
        _main_kernel,
        grid_spec=grid_spec,
        out_shape=jax.ShapeDtypeStruct((bsz, 2), jnp.float32),
    )(cm, x, img_W1, img_b1, img_W2, img_b2, img_W3, img_b3,
      mod_W1, mod_b1, mod_W2, mod_b2,
      inp_emb1, inp_emb2, inp_emb3, loc_emb1, loc_emb2)
    return out


# speedup vs baseline: 2.2880x; 2.2880x over previous
import jax
import jax.numpy as jnp
from jax.experimental import pallas as pl
from jax.experimental.pallas import tpu as pltpu

F32 = jnp.float32


def _first_argmax_mask(y):
    n = y.shape[0]
    it = jax.lax.broadcasted_iota(jnp.int32, y.shape, 0)
    m1 = jnp.max(y, axis=0, keepdims=True)
    a = jnp.min(jnp.where(y == m1, it, n), axis=0, keepdims=True)
    return (it == a).astype(F32)


def _top2_masks(y):
    n = y.shape[0]
    it = jax.lax.broadcasted_iota(jnp.int32, y.shape, 0)
    m1 = jnp.max(y, axis=0, keepdims=True)
    a = jnp.min(jnp.where(y == m1, it, n), axis=0, keepdims=True)
    h1 = (it == a).astype(F32)
    y2 = jnp.where(it == a, -jnp.inf, y)
    m2 = jnp.max(y2, axis=0, keepdims=True)
    b = jnp.min(jnp.where(y2 == m2, it, n), axis=0, keepdims=True)
    h2 = (it == b).astype(F32)
    return h1, h2


def _row_mask(m, n):
    it = jax.lax.broadcasted_iota(jnp.int32, (n, 1), 0)
    return (it == m).astype(F32)


def _routing_kernel(inp0_ref, loc0_ref, out_ref):
    e = inp0_ref[0]
    it = jax.lax.broadcasted_iota(jnp.int32, e.shape, 0)
    mx = jnp.max(e, axis=0, keepdims=True)
    c = jnp.min(jnp.where(e == mx, it, e.shape[0]), axis=0, keepdims=True)
    l = loc0_ref[0]
    it2 = jax.lax.broadcasted_iota(jnp.int32, l.shape, 0)
    mx2 = jnp.max(l, axis=0, keepdims=True)
    m = jnp.min(jnp.where(l == mx2, it2, l.shape[0]), axis=0, keepdims=True)
    out_ref[:] = jnp.concatenate([c, m], axis=0)


def _module_mlp(v1, v2, pm, mw1, mb1, mw2, mb2):
    pm3 = pm[:, :, None]
    w1s = jnp.sum(mw1 * pm3, axis=0)
    b1s = jnp.sum(mb1 * pm, axis=0, keepdims=True)
    w2s = jnp.sum(mw2 * pm3, axis=0)
    b2s = jnp.sum(mb2 * pm, axis=0, keepdims=True)
    h = jnp.maximum(v1 * w1s[0:1, :] + v2 * w1s[1:2, :] + b1s, 0.0)
    return jnp.dot(h, w2s, preferred_element_type=F32) + b2s


def _main_kernel(cm_ref, x_hbm, w1_ref, b1_ref, w2_ref, b2_ref, w3_ref, b3_ref,
                 mw1_ref, mb1_ref, mw2_ref, mb2_ref,
                 ie1_ref, ie2_ref, ie3_ref, le1_ref, le2_ref,
                 out_ref, acc_ref, xbuf_ref, sem):
    i = pl.program_id(0)
    bsz = xbuf_ref.shape[1]

    def copy_for(step, buf):
        c = cm_ref[step]
        return pltpu.make_async_copy(x_hbm.at[:, c, :], xbuf_ref.at[buf], sem.at[buf])

    @pl.when(i == 0)
    def _():
        copy_for(0, 0).start()

    @pl.when(i < 3)
    def _():
        copy_for(i + 1, (i + 1) % 2).start()

    @pl.when(i < 4)
    def _():
        m = cm_ref[4 + i]
        bm = _row_mask(m, 4)
        b1s = jnp.sum(b1_ref[:] * bm, axis=0, keepdims=True)
        b2s = jnp.sum(b2_ref[:] * bm, axis=0, keepdims=True)
        b3s = jnp.sum(b3_ref[:] * bm, axis=0, keepdims=True)
        buf = jax.lax.rem(i, 2)
        copy_for(i, buf).wait()
        flat = xbuf_ref[buf]
        h1 = jnp.maximum(
            jnp.dot(flat, w1_ref[0], preferred_element_type=F32) + b1s, 0.0)
        h2 = jnp.maximum(
            jnp.dot(h1, w2_ref[0], preferred_element_type=F32) + b2s, 0.0)
        y = jnp.dot(h2, w3_ref[0], preferred_element_type=F32) + b3s
        acc_ref[pl.ds(i, 1)] = y.reshape(1, bsz, 16)

    @pl.when(i == 4)
    def _():
        acc = acc_ref[:]
        ie1 = ie1_ref[0]
        le1 = le1_ref[0]
        mw1 = mw1_ref[:]
        mb1 = mb1_ref[:]
        mw2 = mw2_ref[:]
        mb2 = mb2_ref[:]
        cols2 = []
        for si in range(4):
            h1m, h2m = _top2_masks(jax.nn.sigmoid(ie1[:, si:si + 1]))
            h1r = h1m.reshape(4, 16)[:, None, :]
            h2r = h2m.reshape(4, 16)[:, None, :]
            v1 = jnp.sum(acc * h1r, axis=(0, 2))[:, None]
            v2 = jnp.sum(acc * h2r, axis=(0, 2))[:, None]
            pm = _first_argmax_mask(le1[:, si:si + 1])
            cols2.append(_module_mlp(v1, v2, pm, mw1, mb1, mw2, mb2))
        xc2 = jnp.concatenate(cols2, axis=1)
        ie2 = ie2_ref[0]
        le2 = le2_ref[0]
        cols3 = []
        for si in range(2):
            h1m, h2m = _top2_masks(jax.nn.sigmoid(ie2[:, si:si + 1]))
            v1 = jnp.sum(xc2 * h1m.reshape(1, 4), axis=1, keepdims=True)
            v2 = jnp.sum(xc2 * h2m.reshape(1, 4), axis=1, keepdims=True)
            pm = _first_argmax_mask(le2[:, si:si + 1])
            cols3.append(_module_mlp(v1, v2, pm, mw1, mb1, mw2, mb2))
        xc3 = jnp.concatenate(cols3, axis=1)
        h1m, h2m = _top2_masks(jax.nn.sigmoid(ie3_ref[0]))
        v1 = jnp.sum(xc3 * h1m.reshape(1, 2), axis=1, keepdims=True)
        v2 = jnp.sum(xc3 * h2m.reshape(1, 2), axis=1, keepdims=True)
        out_ref[:] = jax.nn.sigmoid(jnp.concatenate([v1, v2], axis=1))


def kernel(x, img_W1, img_b1, img_W2, img_b2, img_W3, img_b3,
           mod_W1, mod_b1, mod_W2, mod_b2,
           inp_emb0, inp_emb1, inp_emb2, inp_emb3,
           loc_emb0, loc_emb1, loc_emb2):
    bsz = x.shape[0]
    cm2 = pl.pallas_call(
        _routing_kernel,
        out_shape=jax.ShapeDtypeStruct((2, 4), jnp.int32),
    )(inp_emb0, loc_emb0)
    cm = cm2.reshape(8)

    def msel(i, cmr):
        return cmr[4 + jnp.minimum(i, 3)]

    def csel(i, cmr):
        return cmr[jnp.minimum(i, 3)]

    grid_spec = pltpu.PrefetchScalarGridSpec(
        num_scalar_prefetch=1,
        grid=(5,),
        in_specs=[
            pl.BlockSpec(memory_space=pl.ANY),
            pl.BlockSpec((1, 784, 512), lambda i, cmr: (msel(i, cmr), 0, 0)),
            pl.BlockSpec((4, 512), lambda i, cmr: (0, 0)),
            pl.BlockSpec((1, 512, 512), lambda i, cmr: (msel(i, cmr), 0, 0)),
            pl.BlockSpec((4, 512), lambda i, cmr: (0, 0)),
            pl.BlockSpec((1, 512, 16), lambda i, cmr: (msel(i, cmr), 0, 0)),
            pl.BlockSpec((4, 16), lambda i, cmr: (0, 0)),
            pl.BlockSpec((8, 2, 128), lambda i, cmr: (0, 0, 0)),
            pl.BlockSpec((8, 128), lambda i, cmr: (0, 0)),
            pl.BlockSpec((8, 128, 1), lambda i, cmr: (0, 0, 0)),
            pl.BlockSpec((8, 1), lambda i, cmr: (0, 0)),
            pl.BlockSpec((1, 64, 4), lambda i, cmr: (0, 0, 0)),
            pl.BlockSpec((1, 4, 2), lambda i, cmr: (0, 0, 0)),
            pl.BlockSpec((1, 2, 1), lambda i, cmr: (0, 0, 0)),
            pl.BlockSpec((1, 8, 4), lambda i, cmr: (0, 0, 0)),
            pl.BlockSpec((1, 8, 2), lambda i, cmr: (0, 0, 0)),
        ],
        out_specs=pl.BlockSpec((bsz, 2), lambda i, cmr: (0, 0)),
        scratch_shapes=[
            pltpu.VMEM((4, bsz, 16), F32),
            pltpu.VMEM((2, bsz, 784), F32),
            pltpu.SemaphoreType.DMA((2,)),
        ],
    )
    out = pl.pallas_call(
        _main_kernel,
        grid_spec=grid_spec,
        out_shape=jax.ShapeDtypeStruct((bsz, 2), jnp.float32),
    )(cm, x.reshape(bsz, 16, 784), img_W1, img_b1, img_W2, img_b2, img_W3, img_b3,
      mod_W1, mod_b1, mod_W2, mod_b2,
      inp_emb1, inp_emb2, inp_emb3, loc_emb1, loc_emb2)
    return out


# speedup vs baseline: 2.4203x; 1.0578x over previous
import jax
import jax.numpy as jnp
from jax.experimental import pallas as pl
from jax.experimental.pallas import tpu as pltpu

F32 = jnp.float32


def _first_argmax_mask(y):
    n = y.shape[0]
    it = jax.lax.broadcasted_iota(jnp.int32, y.shape, 0)
    m1 = jnp.max(y, axis=0, keepdims=True)
    a = jnp.min(jnp.where(y == m1, it, n), axis=0, keepdims=True)
    return (it == a).astype(F32)


def _top2_masks(y):
    n = y.shape[0]
    it = jax.lax.broadcasted_iota(jnp.int32, y.shape, 0)
    m1 = jnp.max(y, axis=0, keepdims=True)
    a = jnp.min(jnp.where(y == m1, it, n), axis=0, keepdims=True)
    h1 = (it == a).astype(F32)
    y2 = jnp.where(it == a, -jnp.inf, y)
    m2 = jnp.max(y2, axis=0, keepdims=True)
    b = jnp.min(jnp.where(y2 == m2, it, n), axis=0, keepdims=True)
    h2 = (it == b).astype(F32)
    return h1, h2


def _row_mask(m, n):
    it = jax.lax.broadcasted_iota(jnp.int32, (n, 1), 0)
    return (it == m).astype(F32)


def _scalar_argmax(ref, n, col):
    bv = ref[0, 0, col]
    bi = jnp.int32(0)
    for k in range(1, n):
        v = ref[0, k, col]
        t = v > bv
        bi = jnp.where(t, jnp.int32(k), bi)
        bv = jnp.where(t, v, bv)
    return bi


def _module_mlp(v1, v2, pm, mw1, mb1, mw2, mb2):
    pm3 = pm[:, :, None]
    w1s = jnp.sum(mw1 * pm3, axis=0)
    b1s = jnp.sum(mb1 * pm, axis=0, keepdims=True)
    w2s = jnp.sum(mw2 * pm3, axis=0)
    b2s = jnp.sum(mb2 * pm, axis=0, keepdims=True)
    h = jnp.maximum(v1 * w1s[0:1, :] + v2 * w1s[1:2, :] + b1s, 0.0)
    return jnp.dot(h, w2s, preferred_element_type=F32) + b2s


def _fused_kernel(ie0_ref, le0_ref, x_hbm, w1_ref, b1_ref, w2_ref, b2_ref,
                  w3_ref, b3_ref, mw1_ref, mb1_ref, mw2_ref, mb2_ref,
                  ie1_ref, ie2_ref, ie3_ref, le1_ref, le2_ref,
                  out_ref, xbuf_ref, sem):
    bsz = xbuf_ref.shape[1]

    cs = [_scalar_argmax(ie0_ref, 16, si) for si in range(4)]
    ms = [_scalar_argmax(le0_ref, 4, si) for si in range(4)]

    def copy_for(i):
        return pltpu.make_async_copy(
            x_hbm.at[:, cs[i], :], xbuf_ref.at[i], sem.at[i])

    for i in range(4):
        copy_for(i).start()

    ys = []
    for i in range(4):
        m = ms[i]
        bm = _row_mask(m, 4)
        b1s = jnp.sum(b1_ref[:] * bm, axis=0, keepdims=True)
        b2s = jnp.sum(b2_ref[:] * bm, axis=0, keepdims=True)
        b3s = jnp.sum(b3_ref[:] * bm, axis=0, keepdims=True)
        copy_for(i).wait()
        flat = xbuf_ref[i]
        h1 = jnp.maximum(
            jnp.dot(flat, w1_ref[m], preferred_element_type=F32) + b1s, 0.0)
        h2 = jnp.maximum(
            jnp.dot(h1, w2_ref[m], preferred_element_type=F32) + b2s, 0.0)
        ys.append(jnp.dot(h2, w3_ref[m], preferred_element_type=F32) + b3s)
    xcat = jnp.concatenate(ys, axis=1)

    ie1 = ie1_ref[0]
    le1 = le1_ref[0]
    mw1 = mw1_ref[:]
    mb1 = mb1_ref[:]
    mw2 = mw2_ref[:]
    mb2 = mb2_ref[:]
    cols2 = []
    for si in range(4):
        h1m, h2m = _top2_masks(jax.nn.sigmoid(ie1[:, si:si + 1]))
        v1 = jnp.sum(xcat * h1m.reshape(1, 64), axis=1, keepdims=True)
        v2 = jnp.sum(xcat * h2m.reshape(1, 64), axis=1, keepdims=True)
        pm = _first_argmax_mask(le1[:, si:si + 1])
        cols2.append(_module_mlp(v1, v2, pm, mw1, mb1, mw2, mb2))
    xc2 = jnp.concatenate(cols2, axis=1)

    ie2 = ie2_ref[0]
    le2 = le2_ref[0]
    cols3 = []
    for si in range(2):
        h1m, h2m = _top2_masks(jax.nn.sigmoid(ie2[:, si:si + 1]))
        v1 = jnp.sum(xc2 * h1m.reshape(1, 4), axis=1, keepdims=True)
        v2 = jnp.sum(xc2 * h2m.reshape(1, 4), axis=1, keepdims=True)
        pm = _first_argmax_mask(le2[:, si:si + 1])
        cols3.append(_module_mlp(v1, v2, pm, mw1, mb1, mw2, mb2))
    xc3 = jnp.concatenate(cols3, axis=1)

    h1m, h2m = _top2_masks(jax.nn.sigmoid(ie3_ref[0]))
    v1 = jnp.sum(xc3 * h1m.reshape(1, 2), axis=1, keepdims=True)
    v2 = jnp.sum(xc3 * h2m.reshape(1, 2), axis=1, keepdims=True)
    out_ref[:] = jax.nn.sigmoid(jnp.concatenate([v1, v2], axis=1))


def kernel(x, img_W1, img_b1, img_W2, img_b2, img_W3, img_b3,
           mod_W1, mod_b1, mod_W2, mod_b2,
           inp_emb0, inp_emb1, inp_emb2, inp_emb3,
           loc_emb0, loc_emb1, loc_emb2):
    bsz = x.shape[0]
    out = pl.pallas_call(
        _fused_kernel,
        in_specs=[
            pl.BlockSpec(memory_space=pltpu.SMEM),
            pl.BlockSpec(memory_space=pltpu.SMEM),
            pl.BlockSpec(memory_space=pl.ANY),
            pl.BlockSpec((4, 784, 512), lambda: (0, 0, 0)),
            pl.BlockSpec((4, 512), lambda: (0, 0)),
            pl.BlockSpec((4, 512, 512), lambda: (0, 0, 0)),
            pl.BlockSpec((4, 512), lambda: (0, 0)),
            pl.BlockSpec((4, 512, 16), lambda: (0, 0, 0)),
            pl.BlockSpec((4, 16), lambda: (0, 0)),
            pl.BlockSpec((8, 2, 128), lambda: (0, 0, 0)),
            pl.BlockSpec((8, 128), lambda: (0, 0)),
            pl.BlockSpec((8, 128, 1), lambda: (0, 0, 0)),
            pl.BlockSpec((8, 1), lambda: (0, 0)),
            pl.BlockSpec((1, 64, 4), lambda: (0, 0, 0)),
            pl.BlockSpec((1, 4, 2), lambda: (0, 0, 0)),
            pl.BlockSpec((1, 2, 1), lambda: (0, 0, 0)),
            pl.BlockSpec((1, 8, 4), lambda: (0, 0, 0)),
            pl.BlockSpec((1, 8, 2), lambda: (0, 0, 0)),
        ],
        out_specs=pl.BlockSpec((bsz, 2), lambda: (0, 0)),
        scratch_shapes=[
            pltpu.VMEM((4, bsz, 784), F32),
            pltpu.SemaphoreType.DMA((4,)),
        ],
        out_shape=jax.ShapeDtypeStruct((bsz, 2), jnp.float32),
    )(inp_emb0, loc_emb0, x.reshape(bsz, 16, 784),
      img_W1, img_b1, img_W2, img_b2, img_W3, img_b3,
      mod_W1, mod_b1, mod_W2, mod_b2,
      inp_emb1, inp_emb2, inp_emb3, loc_emb1, loc_emb2)
    return out


# speedup vs baseline: 2.4253x; 1.0021x over previous
import jax
import jax.numpy as jnp
from jax.experimental import pallas as pl
from jax.experimental.pallas import tpu as pltpu

F32 = jnp.float32


def _first_argmax_mask(y):
    n = y.shape[0]
    it = jax.lax.broadcasted_iota(jnp.int32, y.shape, 0)
    m1 = jnp.max(y, axis=0, keepdims=True)
    a = jnp.min(jnp.where(y == m1, it, n), axis=0, keepdims=True)
    return (it == a).astype(F32)


def _top2_masks(y):
    n = y.shape[0]
    it = jax.lax.broadcasted_iota(jnp.int32, y.shape, 0)
    m1 = jnp.max(y, axis=0, keepdims=True)
    a = jnp.min(jnp.where(y == m1, it, n), axis=0, keepdims=True)
    h1 = (it == a).astype(F32)
    y2 = jnp.where(it == a, -jnp.inf, y)
    m2 = jnp.max(y2, axis=0, keepdims=True)
    b = jnp.min(jnp.where(y2 == m2, it, n), axis=0, keepdims=True)
    h2 = (it == b).astype(F32)
    return h1, h2


def _row_mask(m, n):
    it = jax.lax.broadcasted_iota(jnp.int32, (n, 1), 0)
    return (it == m).astype(F32)


def _scalar_argmax(ref, n, col):
    bv = ref[0, 0, col]
    bi = jnp.int32(0)
    for k in range(1, n):
        v = ref[0, k, col]
        t = v > bv
        bi = jnp.where(t, jnp.int32(k), bi)
        bv = jnp.where(t, v, bv)
    return bi


def _module_mlp(v1, v2, pm, mw1, mb1, mw2, mb2):
    pm3 = pm[:, :, None]
    w1s = jnp.sum(mw1 * pm3, axis=0)
    b1s = jnp.sum(mb1 * pm, axis=0, keepdims=True)
    w2s = jnp.sum(mw2 * pm3, axis=0)
    b2s = jnp.sum(mb2 * pm, axis=0, keepdims=True)
    h = jnp.maximum(v1 * w1s[0:1, :] + v2 * w1s[1:2, :] + b1s, 0.0)
    return jnp.dot(h, w2s, preferred_element_type=F32) + b2s


def _fused_kernel(ie0_ref, le0_ref, x_hbm, w1_ref, b1_ref, w2_ref, b2_ref,
                  w3_ref, b3_ref, mw1_ref, mb1_ref, mw2_ref, mb2_ref,
                  ie1_ref, ie2_ref, ie3_ref, le1_ref, le2_ref,
                  out_ref, xbuf_ref, sem):
    bsz = xbuf_ref.shape[1]

    cs = [_scalar_argmax(ie0_ref, 16, si) for si in range(4)]
    ms = [_scalar_argmax(le0_ref, 4, si) for si in range(4)]

    def copy_for(i):
        return pltpu.make_async_copy(
            x_hbm.at[:, cs[i], :], xbuf_ref.at[i], sem.at[i])

    for i in range(4):
        copy_for(i).start()

    ys = []
    for i in range(4):
        m = ms[i]
        bm = _row_mask(m, 4)
        b1s = jnp.sum(b1_ref[:] * bm, axis=0, keepdims=True)
        b2s = jnp.sum(b2_ref[:] * bm, axis=0, keepdims=True)
        b3s = jnp.sum(b3_ref[:] * bm, axis=0, keepdims=True)
        copy_for(i).wait()
        flat = xbuf_ref[i].astype(jnp.bfloat16)
        h1 = jnp.maximum(
            jnp.dot(flat, w1_ref[m].astype(jnp.bfloat16),
                    preferred_element_type=F32) + b1s, 0.0)
        h2 = jnp.maximum(
            jnp.dot(h1.astype(jnp.bfloat16), w2_ref[m].astype(jnp.bfloat16),
                    preferred_element_type=F32) + b2s, 0.0)
        ys.append(jnp.dot(h2.astype(jnp.bfloat16),
                          w3_ref[m].astype(jnp.bfloat16),
                          preferred_element_type=F32) + b3s)
    xcat = jnp.concatenate(ys, axis=1)

    ie1 = ie1_ref[0]
    le1 = le1_ref[0]
    mw1 = mw1_ref[:]
    mb1 = mb1_ref[:]
    mw2 = mw2_ref[:]
    mb2 = mb2_ref[:]
    hs = []
    for si in range(4):
        h1m, h2m = _top2_masks(jax.nn.sigmoid(ie1[:, si:si + 1]))
        hs.extend([h1m, h2m])
    hmat = jnp.concatenate(hs, axis=1)
    vmat = jnp.dot(xcat, hmat, preferred_element_type=F32)
    cols2 = []
    for si in range(4):
        v1 = vmat[:, 2 * si:2 * si + 1]
        v2 = vmat[:, 2 * si + 1:2 * si + 2]
        pm = _first_argmax_mask(le1[:, si:si + 1])
        cols2.append(_module_mlp(v1, v2, pm, mw1, mb1, mw2, mb2))
    xc2 = jnp.concatenate(cols2, axis=1)

    ie2 = ie2_ref[0]
    le2 = le2_ref[0]
    cols3 = []
    for si in range(2):
        h1m, h2m = _top2_masks(jax.nn.sigmoid(ie2[:, si:si + 1]))
        v1 = jnp.sum(xc2 * h1m.reshape(1, 4), axis=1, keepdims=True)
        v2 = jnp.sum(xc2 * h2m.reshape(1, 4), axis=1, keepdims=True)
        pm = _first_argmax_mask(le2[:, si:si + 1])
        cols3.append(_module_mlp(v1, v2, pm, mw1, mb1, mw2, mb2))
    xc3 = jnp.concatenate(cols3, axis=1)

    h1m, h2m = _top2_masks(jax.nn.sigmoid(ie3_ref[0]))
    v1 = jnp.sum(xc3 * h1m.reshape(1, 2), axis=1, keepdims=True)
    v2 = jnp.sum(xc3 * h2m.reshape(1, 2), axis=1, keepdims=True)
    out_ref[:] = jax.nn.sigmoid(jnp.concatenate([v1, v2], axis=1))


def kernel(x, img_W1, img_b1, img_W2, img_b2, img_W3, img_b3,
           mod_W1, mod_b1, mod_W2, mod_b2,
           inp_emb0, inp_emb1, inp_emb2, inp_emb3,
           loc_emb0, loc_emb1, loc_emb2):
    bsz = x.shape[0]
    out = pl.pallas_call(
        _fused_kernel,
        in_specs=[
            pl.BlockSpec(memory_space=pltpu.SMEM),
            pl.BlockSpec(memory_space=pltpu.SMEM),
            pl.BlockSpec(memory_space=pl.ANY),
            pl.BlockSpec((4, 784, 512), lambda: (0, 0, 0)),
            pl.BlockSpec((4, 512), lambda: (0, 0)),
            pl.BlockSpec((4, 512, 512), lambda: (0, 0, 0)),
            pl.BlockSpec((4, 512), lambda: (0, 0)),
            pl.BlockSpec((4, 512, 16), lambda: (0, 0, 0)),
            pl.BlockSpec((4, 16), lambda: (0, 0)),
            pl.BlockSpec((8, 2, 128), lambda: (0, 0, 0)),
            pl.BlockSpec((8, 128), lambda: (0, 0)),
            pl.BlockSpec((8, 128, 1), lambda: (0, 0, 0)),
            pl.BlockSpec((8, 1), lambda: (0, 0)),
            pl.BlockSpec((1, 64, 4), lambda: (0, 0, 0)),
            pl.BlockSpec((1, 4, 2), lambda: (0, 0, 0)),
            pl.BlockSpec((1, 2, 1), lambda: (0, 0, 0)),
            pl.BlockSpec((1, 8, 4), lambda: (0, 0, 0)),
            pl.BlockSpec((1, 8, 2), lambda: (0, 0, 0)),
        ],
        out_specs=pl.BlockSpec((bsz, 2), lambda: (0, 0)),
        scratch_shapes=[
            pltpu.VMEM((4, bsz, 784), F32),
            pltpu.SemaphoreType.DMA((4,)),
        ],
        out_shape=jax.ShapeDtypeStruct((bsz, 2), jnp.float32),
    )(inp_emb0, loc_emb0, x.reshape(bsz, 16, 784),
      img_W1, img_b1, img_W2, img_b2, img_W3, img_b3,
      mod_W1, mod_b1, mod_W2, mod_b2,
      inp_emb1, inp_emb2, inp_emb3, loc_emb1, loc_emb2)
    return out


# speedup vs baseline: 2.5394x; 1.0470x over previous
import jax
import jax.numpy as jnp
from jax.experimental import pallas as pl
from jax.experimental.pallas import tpu as pltpu

F32 = jnp.float32


def _first_argmax_mask(y):
    n = y.shape[0]
    it = jax.lax.broadcasted_iota(jnp.int32, y.shape, 0)
    m1 = jnp.max(y, axis=0, keepdims=True)
    a = jnp.min(jnp.where(y == m1, it, n), axis=0, keepdims=True)
    return (it == a).astype(F32)


def _top2_masks(y):
    n = y.shape[0]
    it = jax.lax.broadcasted_iota(jnp.int32, y.shape, 0)
    m1 = jnp.max(y, axis=0, keepdims=True)
    a = jnp.min(jnp.where(y == m1, it, n), axis=0, keepdims=True)
    h1 = (it == a).astype(F32)
    y2 = jnp.where(it == a, -jnp.inf, y)
    m2 = jnp.max(y2, axis=0, keepdims=True)
    b = jnp.min(jnp.where(y2 == m2, it, n), axis=0, keepdims=True)
    h2 = (it == b).astype(F32)
    return h1, h2


def _row_mask(m, n):
    it = jax.lax.broadcasted_iota(jnp.int32, (n, 1), 0)
    return (it == m).astype(F32)


def _scalar_argmax(ref, n, col):
    bv = ref[0, 0, col]
    bi = jnp.int32(0)
    for k in range(1, n):
        v = ref[0, k, col]
        t = v > bv
        bi = jnp.where(t, jnp.int32(k), bi)
        bv = jnp.where(t, v, bv)
    return bi


def _module_mlp(v1, v2, pm, mw1, mb1, mw2, mb2):
    pm3 = pm[:, :, None]
    w1s = jnp.sum(mw1 * pm3, axis=0)
    b1s = jnp.sum(mb1 * pm, axis=0, keepdims=True)
    w2s = jnp.sum(mw2 * pm3, axis=0)
    b2s = jnp.sum(mb2 * pm, axis=0, keepdims=True)
    h = jnp.maximum(v1 * w1s[0:1, :] + v2 * w1s[1:2, :] + b1s, 0.0)
    return jnp.dot(h, w2s, preferred_element_type=F32) + b2s


def _fused_kernel(ie0_ref, le0_ref, x_hbm, w1_hbm, b1_ref, w2_hbm, b2_ref,
                  w3_hbm, b3_ref, mw1_ref, mb1_ref, mw2_ref, mb2_ref,
                  ie1_ref, ie2_ref, ie3_ref, le1_ref, le2_ref,
                  out_ref, xbuf_ref, w1_ref, w2_ref, w3_ref, sem, wsem):
    bsz = xbuf_ref.shape[1]

    cs = [_scalar_argmax(ie0_ref, 16, si) for si in range(4)]
    ms = [_scalar_argmax(le0_ref, 4, si) for si in range(4)]

    def copy_for(i):
        return pltpu.make_async_copy(
            x_hbm.at[:, cs[i], :], xbuf_ref.at[i], sem.at[i])

    def wcopy(i, k, src, dst):
        return pltpu.make_async_copy(
            src.at[ms[i]], dst.at[i], wsem.at[3 * i + k])

    for i in range(4):
        copy_for(i).start()
        wcopy(i, 0, w1_hbm, w1_ref).start()
        wcopy(i, 1, w2_hbm, w2_ref).start()
        wcopy(i, 2, w3_hbm, w3_ref).start()

    ys = []
    for i in range(4):
        m = ms[i]
        bm = _row_mask(m, 4)
        b1s = jnp.sum(b1_ref[:] * bm, axis=0, keepdims=True)
        b2s = jnp.sum(b2_ref[:] * bm, axis=0, keepdims=True)
        b3s = jnp.sum(b3_ref[:] * bm, axis=0, keepdims=True)
        copy_for(i).wait()
        wcopy(i, 0, w1_hbm, w1_ref).wait()
        wcopy(i, 1, w2_hbm, w2_ref).wait()
        wcopy(i, 2, w3_hbm, w3_ref).wait()
        flat = xbuf_ref[i].astype(jnp.bfloat16)
        h1 = jnp.maximum(
            jnp.dot(flat, w1_ref[i].astype(jnp.bfloat16),
                    preferred_element_type=F32) + b1s, 0.0)
        h2 = jnp.maximum(
            jnp.dot(h1.astype(jnp.bfloat16), w2_ref[i].astype(jnp.bfloat16),
                    preferred_element_type=F32) + b2s, 0.0)
        ys.append(jnp.dot(h2.astype(jnp.bfloat16),
                          w3_ref[i].astype(jnp.bfloat16),
                          preferred_element_type=F32) + b3s)
    xcat = jnp.concatenate(ys, axis=1)

    ie1 = ie1_ref[0]
    le1 = le1_ref[0]
    mw1 = mw1_ref[:]
    mb1 = mb1_ref[:]
    mw2 = mw2_ref[:]
    mb2 = mb2_ref[:]
    hs = []
    for si in range(4):
        h1m, h2m = _top2_masks(jax.nn.sigmoid(ie1[:, si:si + 1]))
        hs.extend([h1m, h2m])
    hmat = jnp.concatenate(hs, axis=1)
    vmat = jnp.dot(xcat, hmat, preferred_element_type=F32)
    cols2 = []
    for si in range(4):
        v1 = vmat[:, 2 * si:2 * si + 1]
        v2 = vmat[:, 2 * si + 1:2 * si + 2]
        pm = _first_argmax_mask(le1[:, si:si + 1])
        cols2.append(_module_mlp(v1, v2, pm, mw1, mb1, mw2, mb2))
    xc2 = jnp.concatenate(cols2, axis=1)

    ie2 = ie2_ref[0]
    le2 = le2_ref[0]
    cols3 = []
    for si in range(2):
        h1m, h2m = _top2_masks(jax.nn.sigmoid(ie2[:, si:si + 1]))
        v1 = jnp.sum(xc2 * h1m.reshape(1, 4), axis=1, keepdims=True)
        v2 = jnp.sum(xc2 * h2m.reshape(1, 4), axis=1, keepdims=True)
        pm = _first_argmax_mask(le2[:, si:si + 1])
        cols3.append(_module_mlp(v1, v2, pm, mw1, mb1, mw2, mb2))
    xc3 = jnp.concatenate(cols3, axis=1)

    h1m, h2m = _top2_masks(jax.nn.sigmoid(ie3_ref[0]))
    v1 = jnp.sum(xc3 * h1m.reshape(1, 2), axis=1, keepdims=True)
    v2 = jnp.sum(xc3 * h2m.reshape(1, 2), axis=1, keepdims=True)
    out_ref[:] = jax.nn.sigmoid(jnp.concatenate([v1, v2], axis=1))


def kernel(x, img_W1, img_b1, img_W2, img_b2, img_W3, img_b3,
           mod_W1, mod_b1, mod_W2, mod_b2,
           inp_emb0, inp_emb1, inp_emb2, inp_emb3,
           loc_emb0, loc_emb1, loc_emb2):
    bsz = x.shape[0]
    out = pl.pallas_call(
        _fused_kernel,
        in_specs=[
            pl.BlockSpec(memory_space=pltpu.SMEM),
            pl.BlockSpec(memory_space=pltpu.SMEM),
            pl.BlockSpec(memory_space=pl.ANY),
            pl.BlockSpec(memory_space=pl.ANY),
            pl.BlockSpec((4, 512), lambda: (0, 0)),
            pl.BlockSpec(memory_space=pl.ANY),
            pl.BlockSpec((4, 512), lambda: (0, 0)),
            pl.BlockSpec(memory_space=pl.ANY),
            pl.BlockSpec((4, 16), lambda: (0, 0)),
            pl.BlockSpec((8, 2, 128), lambda: (0, 0, 0)),
            pl.BlockSpec((8, 128), lambda: (0, 0)),
            pl.BlockSpec((8, 128, 1), lambda: (0, 0, 0)),
            pl.BlockSpec((8, 1), lambda: (0, 0)),
            pl.BlockSpec((1, 64, 4), lambda: (0, 0, 0)),
            pl.BlockSpec((1, 4, 2), lambda: (0, 0, 0)),
            pl.BlockSpec((1, 2, 1), lambda: (0, 0, 0)),
            pl.BlockSpec((1, 8, 4), lambda: (0, 0, 0)),
            pl.BlockSpec((1, 8, 2), lambda: (0, 0, 0)),
        ],
        out_specs=pl.BlockSpec((bsz, 2), lambda: (0, 0)),
        scratch_shapes=[
            pltpu.VMEM((4, bsz, 784), F32),
            pltpu.VMEM((4, 784, 512), F32),
            pltpu.VMEM((4, 512, 512), F32),
            pltpu.VMEM((4, 512, 16), F32),
            pltpu.SemaphoreType.DMA((4,)),
            pltpu.SemaphoreType.DMA((12,)),
        ],
        out_shape=jax.ShapeDtypeStruct((bsz, 2), jnp.float32),
    )(inp_emb0, loc_emb0, x.reshape(bsz, 16, 784),
      img_W1, img_b1, img_W2, img_b2, img_W3, img_b3,
      mod_W1, mod_b1, mod_W2, mod_b2,
      inp_emb1, inp_emb2, inp_emb3, loc_emb1, loc_emb2)
    return out
